# Initial kernel scaffold; baseline (speedup 1.0000x reference)
import jax
import jax.numpy as jnp
from jax.experimental import pallas as pl


def kernel(inputs, table):
    raise NotImplementedError("write your pallas kernel here")


# speedup vs baseline: 2.0073x; 2.0073x over previous
import functools

import jax
import jax.numpy as jnp
from jax import lax
from jax.experimental import pallas as pl
from jax.experimental.pallas import tpu as pltpu
from jax.experimental.pallas import tpu_sc as plsc

VOCAB_DIM = 100000
D = 128
B = 16384
C = 4
L = 16
NC = 2
NS = 16
NW = NC * NS
B_PER_W = B // NW
CHUNK = 32
N_CHUNKS = B_PER_W // CHUNK
N_PAIRS = N_CHUNKS // 2


def _cbow_kernel(idx_hbm, table_hbm, out_hbm, idx_v, rows_v, out_v,
                 sem0, sem1, idx_sem):
    wid = lax.axis_index("s") * NC + lax.axis_index("c")
    base = wid * B_PER_W

    pltpu.async_copy(idx_hbm.at[wid], idx_v, idx_sem).wait()

    def gather(ci, buf, sem):
        return pltpu.make_async_copy(
            table_hbm.at[idx_v.at[ci]], rows_v.at[buf], sem)

    def compute(ci, buf):
        def row_body(b, _):
            rb = b * C
            for d in range(D // L):
                ds = pl.ds(d * L, L)
                s = ((rows_v[buf, rb, ds] + rows_v[buf, rb + 1, ds])
                     + (rows_v[buf, rb + 2, ds] + rows_v[buf, rb + 3, ds]))
                out_v[buf, b, ds] = s * (1.0 / C)
            return _

        lax.fori_loop(0, CHUNK, row_body, 0, unroll=4)
        pltpu.sync_copy(out_v.at[buf],
                        out_hbm.at[pl.ds(base + ci * CHUNK, CHUNK)])

    gather(0, 0, sem0).start()

    def pair_body(t, _):
        ci0 = t * 2
        g1 = gather(ci0 + 1, 1, sem1)
        g1.start()
        gather(ci0, 0, sem0).wait()
        compute(ci0, 0)
        nci = lax.min(ci0 + 2, N_CHUNKS - 1)
        gather(nci, 0, sem0).start()
        g1.wait()
        compute(ci0 + 1, 1)
        return _

    lax.fori_loop(0, N_PAIRS, pair_body, 0)
    gather(N_CHUNKS - 1, 0, sem0).wait()


@jax.jit
def _cbow(idx_flat, table):
    mesh = plsc.VectorSubcoreMesh(core_axis_name="c", subcore_axis_name="s")
    kern = functools.partial(
        pl.kernel,
        mesh=mesh,
        out_type=jax.ShapeDtypeStruct((B, D), jnp.float32),
        scratch_types=[
            pltpu.VMEM((N_CHUNKS, CHUNK * C), jnp.int32),
            pltpu.VMEM((2, CHUNK * C, D), jnp.float32),
            pltpu.VMEM((2, CHUNK, D), jnp.float32),
            pltpu.SemaphoreType.DMA,
            pltpu.SemaphoreType.DMA,
            pltpu.SemaphoreType.DMA,
        ],
    )(_cbow_kernel)
    return kern(idx_flat, table)


def kernel(inputs, table):
    idx_flat = inputs.astype(jnp.int32).reshape(NW, N_CHUNKS, CHUNK * C)
    return _cbow(idx_flat, table)
